# baseline (speedup 1.0000x reference)
import jax
import jax.numpy as jnp
from jax import lax
from jax.experimental import pallas as pl
from jax.experimental.pallas import tpu as pltpu
from jax.experimental.pallas import tpu_sc as plsc

VOCAB = 100000
EMBED_DIM = 128
MAX_SEQ = 4096
BATCH = 4
SEQ = 4096

_INFO = plsc.get_sparse_core_info()
NC = _INFO.num_cores
NS = _INFO.num_subcores
L = _INFO.num_lanes
NW = NC * NS
SPW = SEQ // NW
LANESETS = EMBED_DIM // L


NCH = 2
CW = SPW // NCH
NCHUNK = BATCH * NCH
NE = 2


def _body(ids_hbm, table_hbm, pos_hbm, out_hbm, idx_v, pos_v, tok_v,
          isem, psem, osem, *gsems):
    wid = lax.axis_index("s") * NC + lax.axis_index("c")
    s0 = wid * SPW

    idx_cp = pltpu.async_copy(ids_hbm.at[:, pl.ds(s0, SPW)], idx_v, isem)
    pos_cp = pltpu.async_copy(pos_hbm.at[pl.ds(s0, SPW)], pos_v, psem)
    idx_cp.wait()

    gcps = [
        pltpu.async_copy(
            table_hbm.at[idx_v.at[c // NCH, pl.ds((c % NCH) * CW, CW)]],
            tok_v.at[pl.ds(c * CW, CW)],
            gsems[c],
        )
        for c in range(NE)
    ]
    pos_cp.wait()

    for c in range(NE, NCHUNK):
        p0 = (c % NCH) * CW

        @plsc.parallel_loop(0, CW, unroll=2)
        def _fill(r, _c=c, _p0=p0):
            for l in range(LANESETS):
                sl = pl.ds(l * L, L)
                tok_v[_c * CW + r, sl] = pos_v[_p0 + r, sl]

        gcps.append(
            pltpu.async_copy(
                table_hbm.at[idx_v.at[c // NCH, pl.ds(p0, CW)]],
                tok_v.at[pl.ds(c * CW, CW)],
                gsems[c],
                add=True,
            )
        )

    ocps = []
    for c in range(NCHUNK):
        gcps[c].wait()
        if c < NE:
            p0 = (c % NCH) * CW

            @plsc.parallel_loop(0, CW, unroll=2)
            def _add0(r, _c=c, _p0=p0):
                for l in range(LANESETS):
                    sl = pl.ds(l * L, L)
                    plsc.addupdate(tok_v.at[_c * CW + r, sl], pos_v[_p0 + r, sl])

        ocps.append(
            pltpu.async_copy(
                tok_v.at[pl.ds(c * CW, CW)],
                out_hbm.at[c // NCH, pl.ds(s0 + (c % NCH) * CW, CW)],
                osem,
            )
        )
    for c in ocps:
        c.wait()


_emb = pl.kernel(
    _body,
    out_type=jax.ShapeDtypeStruct((BATCH, SEQ, EMBED_DIM), jnp.float32),
    mesh=plsc.VectorSubcoreMesh(core_axis_name="c", subcore_axis_name="s"),
    compiler_params=pltpu.CompilerParams(
        skip_device_barrier=True,
        disable_bounds_checks=True,
        disable_semaphore_checks=True,
    ),
    scratch_types=[
        pltpu.VMEM((BATCH, SPW), jnp.int32),
        pltpu.VMEM((SPW, EMBED_DIM), jnp.float32),
        pltpu.VMEM((BATCH * SPW, EMBED_DIM), jnp.float32),
    ] + [pltpu.SemaphoreType.DMA] * (3 + NCHUNK),
)


@jax.jit
def kernel(token_ids, token_table, pos_table):
    return _emb(token_ids.astype(jnp.int32), token_table, pos_table)


# speedup vs baseline: 1.0062x; 1.0062x over previous
import jax
import jax.numpy as jnp
from jax import lax
from jax.experimental import pallas as pl
from jax.experimental.pallas import tpu as pltpu
from jax.experimental.pallas import tpu_sc as plsc

VOCAB = 100000
EMBED_DIM = 128
MAX_SEQ = 4096
BATCH = 4
SEQ = 4096

_INFO = plsc.get_sparse_core_info()
NC = _INFO.num_cores
NS = _INFO.num_subcores
L = _INFO.num_lanes
NW = NC * NS
SPW = SEQ // NW
LANESETS = EMBED_DIM // L


NCH = 2
CW = SPW // NCH
NCHUNK = BATCH * NCH


def _body(ids_hbm, table_hbm, pos_hbm, out_hbm, idx_v, pos_v, tok_v,
          isem, psem, osem, *gsems):
    wid = lax.axis_index("s") * NC + lax.axis_index("c")
    s0 = wid * SPW

    idx_cp = pltpu.async_copy(ids_hbm.at[:, pl.ds(s0, SPW)], idx_v, isem)
    pos_cp = pltpu.async_copy(pos_hbm.at[pl.ds(s0, SPW)], pos_v, psem)
    idx_cp.wait()

    gcps = [
        pltpu.async_copy(
            table_hbm.at[idx_v.at[0, pl.ds(0, CW)]],
            tok_v.at[pl.ds(0, CW)],
            gsems[0],
        )
    ]
    pos_cp.wait()

    for c in range(1, NCHUNK):
        p0 = (c % NCH) * CW

        @plsc.parallel_loop(0, CW, unroll=2)
        def _fill(r, _c=c, _p0=p0):
            for l in range(LANESETS):
                sl = pl.ds(l * L, L)
                tok_v[_c * CW + r, sl] = pos_v[_p0 + r, sl]

        gcps.append(
            pltpu.async_copy(
                table_hbm.at[idx_v.at[c // NCH, pl.ds(p0, CW)]],
                tok_v.at[pl.ds(c * CW, CW)],
                gsems[c],
                add=True,
            )
        )

    ocps = []
    for c in range(NCHUNK):
        gcps[c].wait()
        if c == 0:
            @plsc.parallel_loop(0, CW, unroll=2)
            def _add0(r):
                for l in range(LANESETS):
                    sl = pl.ds(l * L, L)
                    plsc.addupdate(tok_v.at[r, sl], pos_v[r, sl])

        ocps.append(
            pltpu.async_copy(
                tok_v.at[pl.ds(c * CW, CW)],
                out_hbm.at[c // NCH, pl.ds(s0 + (c % NCH) * CW, CW)],
                osem,
            )
        )
    for c in ocps:
        c.wait()


_emb = pl.kernel(
    _body,
    out_type=jax.ShapeDtypeStruct((BATCH, SEQ, EMBED_DIM), jnp.float32),
    mesh=plsc.VectorSubcoreMesh(core_axis_name="c", subcore_axis_name="s"),
    compiler_params=pltpu.CompilerParams(
        skip_device_barrier=True,
        disable_bounds_checks=True,
        disable_semaphore_checks=True,
    ),
    scratch_types=[
        pltpu.VMEM((BATCH, SPW), jnp.int32),
        pltpu.VMEM((SPW, EMBED_DIM), jnp.float32),
        pltpu.VMEM((BATCH * SPW, EMBED_DIM), jnp.float32),
    ] + [pltpu.SemaphoreType.DMA] * (3 + NCHUNK),
)


@jax.jit
def kernel(token_ids, token_table, pos_table):
    return _emb(token_ids.astype(jnp.int32), token_table, pos_table)
